# Initial kernel scaffold; baseline (speedup 1.0000x reference)
import jax
import jax.numpy as jnp
from jax.experimental import pallas as pl


def kernel(test_patches, memory_nodes, W1, b1, W2, b2):
    raise NotImplementedError("write your pallas kernel here")


# speedup vs baseline: 17.8798x; 17.8798x over previous
import functools

import jax
import jax.numpy as jnp
from jax import lax
from jax.experimental import pallas as pl
from jax.experimental.pallas import tpu as pltpu
from jax.experimental.pallas import tpu_sc as plsc

_TOP_K = 50
_LANES = 16


def _sim_matmul_kernel(a_ref, b_ref, o_ref):
    o_ref[...] = lax.dot_general(
        a_ref[...], b_ref[...],
        dimension_numbers=(((1,), (1,)), ((), ())),
        preferred_element_type=jnp.float32,
    )


def _sim_matmul(patches, nodes):
    p, d = patches.shape
    m = nodes.shape[0]
    bm = 1024
    return pl.pallas_call(
        _sim_matmul_kernel,
        grid=(m // bm,),
        in_specs=[
            pl.BlockSpec((p, d), lambda j: (0, 0)),
            pl.BlockSpec((bm, d), lambda j: (j, 0)),
        ],
        out_specs=pl.BlockSpec((p, bm), lambda j: (0, j)),
        out_shape=jax.ShapeDtypeStruct((p, m), jnp.float32),
    )(patches, nodes)


def _topk_sc(sim, k_pad):
    p, m = sim.shape
    nw = 32
    rpw = p // nw
    nv = m // _LANES
    nl1 = nv // _LANES
    nl2 = nl1 // _LANES
    assert nl2 == _LANES

    mesh = plsc.VectorSubcoreMesh(core_axis_name="c", subcore_axis_name="s")

    @functools.partial(
        pl.kernel,
        mesh=mesh,
        compiler_params=pltpu.CompilerParams(needs_layout_passes=False),
        out_type=jax.ShapeDtypeStruct((p, k_pad), jnp.int32),
        scratch_types=[
            pltpu.VMEM((m,), jnp.float32),
            pltpu.VMEM((nv,), jnp.float32),
            pltpu.VMEM((nl1,), jnp.float32),
            pltpu.VMEM((rpw, k_pad), jnp.int32),
        ],
    )
    def topk_kernel(sim_hbm, out_hbm, row_v, l1_v, l2_v, out_v):
        cid = lax.axis_index("c")
        sid = lax.axis_index("s")
        wid = sid * 2 + cid
        base = wid * rpw
        iota = lax.iota(jnp.int32, _LANES)
        lane0 = iota == 0
        neg_inf_v = jnp.full((_LANES,), -jnp.inf, jnp.float32)

        def _full_i(x):
            return jnp.full((_LANES,), x, jnp.int32)

        def _full_f(x):
            return jnp.full((_LANES,), x, jnp.float32)

        def row_body(r, carry):
            pltpu.sync_copy(sim_hbm.at[base + r], row_v)

            def l1_body(i, c):
                mx = row_v[pl.ds(i * 256, _LANES)]
                for j in range(1, _LANES):
                    mx = jnp.maximum(mx, row_v[pl.ds(i * 256 + j * 16, _LANES)])
                l1_v[pl.ds(i * 16, _LANES)] = mx
                return c
            lax.fori_loop(0, nl1, l1_body, 0)

            def l2_body(i, c):
                mx = l1_v[pl.ds(i * 256, _LANES)]
                for j in range(1, _LANES):
                    mx = jnp.maximum(mx, l1_v[pl.ds(i * 256 + j * 16, _LANES)])
                l2_v[pl.ds(i * 16, _LANES)] = mx
                return c
            lax.fori_loop(0, nl2, l2_body, 0)

            l3 = l2_v[pl.ds(0, _LANES)]
            for j in range(1, _LANES):
                l3 = jnp.maximum(l3, l2_v[pl.ds(j * 16, _LANES)])

            def ext_body(e, l3c):
                ks3, is3 = plsc.sort_key_val(l3c, iota, descending=True)
                a3 = is3[0]
                v2 = plsc.load_gather(l2_v, [iota * 16 + a3])
                ks2, is2 = plsc.sort_key_val(v2, iota, descending=True)
                j2 = is2[0]
                v1 = plsc.load_gather(
                    l1_v, [_full_i(j2 * 256 + a3) + iota * 16])
                ks1, is1 = plsc.sort_key_val(v1, iota, descending=True)
                j1 = is1[0]
                i1 = j2 * 16 + j1
                g = plsc.load_gather(
                    row_v, [_full_i(i1 * 256 + a3) + iota * 16])
                ks0, is0 = plsc.sort_key_val(g, iota, descending=True)
                j0 = is0[0]
                pos = i1 * 256 + j0 * 16 + a3

                plsc.store_scatter(
                    out_v, [_full_i(r), _full_i(e)], _full_i(pos), mask=lane0)
                plsc.store_scatter(row_v, [_full_i(pos)], neg_inf_v, mask=lane0)
                new_leaf = ks0[1]
                plsc.store_scatter(
                    l1_v, [_full_i(i1 * 16 + a3)], _full_f(new_leaf), mask=lane0)
                new_l1max = jnp.maximum(ks1[1], new_leaf)
                plsc.store_scatter(
                    l2_v, [_full_i(j2 * 16 + a3)], _full_f(new_l1max), mask=lane0)
                new_l2max = jnp.maximum(ks2[1], new_l1max)
                return jnp.where(iota == a3, new_l2max, l3c)

            lax.fori_loop(0, _TOP_K, ext_body, l3)
            return carry

        lax.fori_loop(0, rpw, row_body, 0)
        pltpu.sync_copy(out_v, out_hbm.at[pl.ds(base, rpw)])

    return topk_kernel(sim)


def _pool_kernel(p_ref, w1_ref, b1_ref, w2t_ref, b2_ref, o_ref):
    patches = p_ref[...]
    h = lax.dot_general(
        patches, w1_ref[...],
        dimension_numbers=(((1,), (0,)), ((), ())),
        preferred_element_type=jnp.float32,
    )
    h = jnp.maximum(h + b1_ref[...], 0.0)
    z = jnp.sum(h * w2t_ref[...], axis=1, keepdims=True) + b2_ref[...]
    z = z - jnp.max(z)
    w = jnp.exp(z)
    w = w / jnp.sum(w)
    gf = jnp.sum(patches * w, axis=0, keepdims=True)
    n = jnp.sqrt(jnp.sum(gf * gf))
    o_ref[...] = gf / jnp.maximum(n, 1e-12)


def _pool(patches, w1, b1, w2, b2):
    p, d = patches.shape
    dh = w1.shape[1]
    return pl.pallas_call(
        _pool_kernel,
        out_shape=jax.ShapeDtypeStruct((1, d), jnp.float32),
    )(patches, w1, b1.reshape(1, dh), w2.reshape(1, dh), b2.reshape(1, 1))


def kernel(test_patches, memory_nodes, W1, b1, W2, b2):
    p = test_patches.shape[0]
    sim = _sim_matmul(test_patches, memory_nodes)
    idx_pad = _topk_sc(sim, 64)
    topk = idx_pad[:, :_TOP_K]

    memory_node_idx = topk.reshape(-1)
    test_node_idx = jnp.broadcast_to(
        jnp.arange(p, dtype=jnp.int32)[:, None], (p, _TOP_K)).reshape(-1)
    edge_index = jnp.stack([memory_node_idx, test_node_idx], axis=0)

    global_feature = _pool(test_patches, W1, b1, W2, b2)
    return edge_index, global_feature


# speedup vs baseline: 21.7917x; 1.2188x over previous
import functools

import jax
import jax.numpy as jnp
from jax import lax
from jax.experimental import pallas as pl
from jax.experimental.pallas import tpu as pltpu
from jax.experimental.pallas import tpu_sc as plsc

_TOP_K = 50
_LANES = 16


def _sim_matmul_kernel(a_ref, b_ref, o_ref):
    o_ref[...] = lax.dot_general(
        a_ref[...], b_ref[...],
        dimension_numbers=(((1,), (1,)), ((), ())),
        preferred_element_type=jnp.float32,
    )


def _sim_matmul(patches, nodes):
    p, d = patches.shape
    m = nodes.shape[0]
    bm = 1024
    return pl.pallas_call(
        _sim_matmul_kernel,
        grid=(m // bm,),
        in_specs=[
            pl.BlockSpec((p, d), lambda j: (0, 0)),
            pl.BlockSpec((bm, d), lambda j: (j, 0)),
        ],
        out_specs=pl.BlockSpec((p, bm), lambda j: (0, j)),
        out_shape=jax.ShapeDtypeStruct((p, m), jnp.float32),
    )(patches, nodes)


def _topk_sc(sim, k_pad):
    p, m = sim.shape
    nw = 32
    rpw = p // nw
    nv = m // _LANES
    nl1 = nv // _LANES
    nl2 = nl1 // _LANES
    assert nl2 == _LANES
    half = m // 2
    hl1 = nl1 // 2
    assert rpw % 3 == 1

    mesh = plsc.VectorSubcoreMesh(core_axis_name="c", subcore_axis_name="s")

    @functools.partial(
        pl.kernel,
        mesh=mesh,
        compiler_params=pltpu.CompilerParams(needs_layout_passes=False),
        out_type=jax.ShapeDtypeStruct((p, k_pad), jnp.int32),
        scratch_types=[
            pltpu.VMEM((3 * half,), jnp.float32),
            pltpu.VMEM((nv,), jnp.float32),
            pltpu.VMEM((nl1,), jnp.float32),
            pltpu.VMEM((rpw, k_pad), jnp.int32),
            pltpu.SemaphoreType.DMA,
            pltpu.SemaphoreType.DMA,
            pltpu.SemaphoreType.DMA,
        ],
    )
    def topk_kernel(sim_hbm, out_hbm, big_v, l1_v, l2_v, out_v, s_a, s_b, s_c):
        sems = (s_a, s_b, s_c)
        cid = lax.axis_index("c")
        sid = lax.axis_index("s")
        wid = sid * 2 + cid
        base = wid * rpw
        iota = lax.iota(jnp.int32, _LANES)
        lane0 = iota == 0
        neg_inf_v = jnp.full((_LANES,), -jnp.inf, jnp.float32)

        def _full_i(x):
            return jnp.full((_LANES,), x, jnp.int32)

        def _full_f(x):
            return jnp.full((_LANES,), x, jnp.float32)

        def _dma(row, h, slot):
            return pltpu.make_async_copy(
                sim_hbm.at[row, pl.ds(h * half, half)],
                big_v.at[pl.ds(slot * half, half)],
                sems[slot])

        def do_row(r, s0, s1, sf, prefetch):
            _dma(base + r, 0, s0).wait()

            def l1a(i, c):
                mx = big_v[pl.ds(s0 * half + i * 256, _LANES)]
                for j in range(1, _LANES):
                    mx = jnp.maximum(
                        mx, big_v[pl.ds(s0 * half + i * 256 + j * 16, _LANES)])
                l1_v[pl.ds(i * 16, _LANES)] = mx
                return c
            lax.fori_loop(0, hl1, l1a, 0)
            _dma(base + r, 1, s1).wait()

            def l1b(i, c):
                mx = big_v[pl.ds(s1 * half + i * 256, _LANES)]
                for j in range(1, _LANES):
                    mx = jnp.maximum(
                        mx, big_v[pl.ds(s1 * half + i * 256 + j * 16, _LANES)])
                l1_v[pl.ds((i + hl1) * 16, _LANES)] = mx
                return c
            lax.fori_loop(0, hl1, l1b, 0)

            def l2_body(i, c):
                mx = l1_v[pl.ds(i * 256, _LANES)]
                for j in range(1, _LANES):
                    mx = jnp.maximum(mx, l1_v[pl.ds(i * 256 + j * 16, _LANES)])
                l2_v[pl.ds(i * 16, _LANES)] = mx
                return c
            lax.fori_loop(0, nl2, l2_body, 0)

            l3 = l2_v[pl.ds(0, _LANES)]
            for j in range(1, _LANES):
                l3 = jnp.maximum(l3, l2_v[pl.ds(j * 16, _LANES)])

            if prefetch:
                _dma(base + r + 1, 0, sf).start()

            def ext_body(e, l3c):
                ks3, is3 = plsc.sort_key_val(l3c, iota, descending=True)
                a3 = is3[0]
                v2 = plsc.load_gather(l2_v, [iota * 16 + a3])
                ks2, is2 = plsc.sort_key_val(v2, iota, descending=True)
                j2 = is2[0]
                v1 = plsc.load_gather(
                    l1_v, [_full_i(j2 * 256 + a3) + iota * 16])
                ks1, is1 = plsc.sort_key_val(v1, iota, descending=True)
                j1 = is1[0]
                i1 = j2 * 16 + j1
                pbase = jnp.where(i1 < hl1,
                                  s0 * half + i1 * 256,
                                  (s1 - 1) * half + i1 * 256) + a3
                g = plsc.load_gather(big_v, [_full_i(pbase) + iota * 16])
                ks0, is0 = plsc.sort_key_val(g, iota, descending=True)
                j0 = is0[0]
                pos = i1 * 256 + j0 * 16 + a3

                plsc.store_scatter(
                    out_v, [_full_i(r), _full_i(e)], _full_i(pos), mask=lane0)
                plsc.store_scatter(
                    big_v, [_full_i(pbase + j0 * 16)], neg_inf_v, mask=lane0)
                new_leaf = ks0[1]
                plsc.store_scatter(
                    l1_v, [_full_i(i1 * 16 + a3)], _full_f(new_leaf), mask=lane0)
                new_l1max = jnp.maximum(ks1[1], new_leaf)
                plsc.store_scatter(
                    l2_v, [_full_i(j2 * 16 + a3)], _full_f(new_l1max), mask=lane0)
                new_l2max = jnp.maximum(ks2[1], new_l1max)
                return jnp.where(iota == a3, new_l2max, l3c)

            lax.fori_loop(0, _TOP_K, ext_body, l3)
            if prefetch:
                _dma(base + r + 1, 1, s0).start()

        _dma(base, 0, 0).start()
        _dma(base, 1, 1).start()

        def tri_body(t, c):
            r = t * 3
            do_row(r, 0, 1, 2, True)
            do_row(r + 1, 2, 0, 1, True)
            do_row(r + 2, 1, 2, 0, True)
            return c
        lax.fori_loop(0, (rpw - 1) // 3, tri_body, 0)
        do_row(rpw - 1, 0, 1, 2, False)
        pltpu.sync_copy(out_v, out_hbm.at[pl.ds(base, rpw)])

    return topk_kernel(sim)


def _pool_kernel(p_ref, w1_ref, b1_ref, w2t_ref, b2_ref, o_ref):
    patches = p_ref[...]
    h = lax.dot_general(
        patches, w1_ref[...],
        dimension_numbers=(((1,), (0,)), ((), ())),
        preferred_element_type=jnp.float32,
    )
    h = jnp.maximum(h + b1_ref[...], 0.0)
    z = jnp.sum(h * w2t_ref[...], axis=1, keepdims=True) + b2_ref[...]
    z = z - jnp.max(z)
    w = jnp.exp(z)
    w = w / jnp.sum(w)
    gf = jnp.sum(patches * w, axis=0, keepdims=True)
    n = jnp.sqrt(jnp.sum(gf * gf))
    o_ref[...] = gf / jnp.maximum(n, 1e-12)


def _pool(patches, w1, b1, w2, b2):
    p, d = patches.shape
    dh = w1.shape[1]
    return pl.pallas_call(
        _pool_kernel,
        out_shape=jax.ShapeDtypeStruct((1, d), jnp.float32),
    )(patches, w1, b1.reshape(1, dh), w2.reshape(1, dh), b2.reshape(1, 1))


def kernel(test_patches, memory_nodes, W1, b1, W2, b2):
    p = test_patches.shape[0]
    sim = _sim_matmul(test_patches, memory_nodes)
    idx_pad = _topk_sc(sim, 64)
    topk = idx_pad[:, :_TOP_K]

    memory_node_idx = topk.reshape(-1)
    test_node_idx = jnp.broadcast_to(
        jnp.arange(p, dtype=jnp.int32)[:, None], (p, _TOP_K)).reshape(-1)
    edge_index = jnp.stack([memory_node_idx, test_node_idx], axis=0)

    global_feature = _pool(test_patches, W1, b1, W2, b2)
    return edge_index, global_feature


# speedup vs baseline: 24.7687x; 1.1366x over previous
import functools

import jax
import jax.numpy as jnp
from jax import lax
from jax.experimental import pallas as pl
from jax.experimental.pallas import tpu as pltpu
from jax.experimental.pallas import tpu_sc as plsc

_TOP_K = 50
_LANES = 16


def _sim_matmul_kernel(a_ref, b_ref, o_ref, l1_ref):
    s = lax.dot_general(
        a_ref[...], b_ref[...],
        dimension_numbers=(((1,), (1,)), ((), ())),
        preferred_element_type=jnp.float32,
    )
    o_ref[...] = s
    mx = s[:, 0:128]
    for q in range(1, _LANES):
        mx = jnp.maximum(mx, s[:, q * 128:(q + 1) * 128])
    l1_ref[...] = mx


def _sim_matmul(patches, nodes):
    p, d = patches.shape
    m = nodes.shape[0]
    bm = 2048
    return pl.pallas_call(
        _sim_matmul_kernel,
        grid=(m // bm,),
        in_specs=[
            pl.BlockSpec((p, d), lambda j: (0, 0)),
            pl.BlockSpec((bm, d), lambda j: (j, 0)),
        ],
        out_specs=[
            pl.BlockSpec((p, bm), lambda j: (0, j)),
            pl.BlockSpec((p, 128), lambda j: (0, j)),
        ],
        out_shape=[
            jax.ShapeDtypeStruct((p, m), jnp.float32),
            jax.ShapeDtypeStruct((p, m // _LANES), jnp.float32),
        ],
    )(patches, nodes)


def _topk_sc(sim, l1, k_pad):
    p, m = sim.shape
    nw = 32
    rpw = p // nw
    nv = m // _LANES
    nl1 = nv // _LANES
    nl2 = nl1 // _LANES
    assert nl2 == _LANES
    half = m // 2
    assert rpw % 3 == 1

    mesh = plsc.VectorSubcoreMesh(core_axis_name="c", subcore_axis_name="s")

    @functools.partial(
        pl.kernel,
        mesh=mesh,
        compiler_params=pltpu.CompilerParams(needs_layout_passes=False),
        out_type=jax.ShapeDtypeStruct((p, k_pad), jnp.int32),
        scratch_types=[
            pltpu.VMEM((3 * half,), jnp.float32),
            pltpu.VMEM((3 * nv,), jnp.float32),
            pltpu.VMEM((nl1,), jnp.float32),
            pltpu.VMEM((rpw, k_pad), jnp.int32),
            pltpu.SemaphoreType.DMA,
            pltpu.SemaphoreType.DMA,
            pltpu.SemaphoreType.DMA,
            pltpu.SemaphoreType.DMA,
            pltpu.SemaphoreType.DMA,
            pltpu.SemaphoreType.DMA,
        ],
    )
    def topk_kernel(sim_hbm, l1_hbm, out_hbm, big_v, l1_v, l2_v, out_v,
                    s_a, s_b, s_c, t_a, t_b, t_c):
        sems = (s_a, s_b, s_c)
        lsems = (t_a, t_b, t_c)
        cid = lax.axis_index("c")
        sid = lax.axis_index("s")
        wid = sid * 2 + cid
        base = wid * rpw
        iota = lax.iota(jnp.int32, _LANES)
        lane0 = iota == 0
        neg_inf_v = jnp.full((_LANES,), -jnp.inf, jnp.float32)

        def _full_i(x):
            return jnp.full((_LANES,), x, jnp.int32)

        def _full_f(x):
            return jnp.full((_LANES,), x, jnp.float32)

        def _dma(row, h, slot):
            return pltpu.make_async_copy(
                sim_hbm.at[row, pl.ds(h * half, half)],
                big_v.at[pl.ds(slot * half, half)],
                sems[slot])

        def _dma_l1(row, slot):
            return pltpu.make_async_copy(
                l1_hbm.at[row],
                l1_v.at[pl.ds(slot * nv, nv)],
                lsems[slot])

        def do_row(r, s0, s1, sf, prefetch):
            lbase = s0 * nv
            _dma_l1(base + r, s0).wait()

            def l2_body(i, c):
                mx = l1_v[pl.ds(lbase + i * 256, _LANES)]
                for j in range(1, _LANES):
                    mx = jnp.maximum(
                        mx, l1_v[pl.ds(lbase + i * 256 + j * 16, _LANES)])
                l2_v[pl.ds(i * 16, _LANES)] = mx
                return c
            lax.fori_loop(0, nl2, l2_body, 0)

            l3 = l2_v[pl.ds(0, _LANES)]
            for j in range(1, _LANES):
                l3 = jnp.maximum(l3, l2_v[pl.ds(j * 16, _LANES)])

            _dma(base + r, 0, s0).wait()
            _dma(base + r, 1, s1).wait()
            if prefetch:
                _dma(base + r + 1, 0, sf).start()
                _dma_l1(base + r + 1, sf).start()

            def ext_body(e, l3c):
                ks3, is3 = plsc.sort_key_val(l3c, iota, descending=True)
                a3 = is3[0]
                v2 = plsc.load_gather(l2_v, [iota * 16 + a3])
                ks2, is2 = plsc.sort_key_val(v2, iota, descending=True)
                j2 = is2[0]
                v1 = plsc.load_gather(
                    l1_v, [_full_i(lbase + j2 * 256 + a3) + iota * 16])
                ks1, is1 = plsc.sort_key_val(v1, iota, descending=True)
                j1 = is1[0]
                q1 = j2 * 256 + j1 * 16 + a3
                b = q1 // 128
                ql = q1 - b * 128
                pbase = jnp.where(b < _LANES,
                                  s0 * half + b * 2048,
                                  (s1 - 1) * half + b * 2048) + ql
                g = plsc.load_gather(big_v, [_full_i(pbase) + iota * 128])
                ks0, is0 = plsc.sort_key_val(g, iota, descending=True)
                j0 = is0[0]
                pos = b * 2048 + j0 * 128 + ql

                plsc.store_scatter(
                    out_v, [_full_i(r), _full_i(e)], _full_i(pos), mask=lane0)
                plsc.store_scatter(
                    big_v, [_full_i(pbase + j0 * 128)], neg_inf_v, mask=lane0)
                new_leaf = ks0[1]
                plsc.store_scatter(
                    l1_v, [_full_i(lbase + q1)], _full_f(new_leaf), mask=lane0)
                new_l1max = jnp.maximum(ks1[1], new_leaf)
                plsc.store_scatter(
                    l2_v, [_full_i(j2 * 16 + a3)], _full_f(new_l1max), mask=lane0)
                new_l2max = jnp.maximum(ks2[1], new_l1max)
                return jnp.where(iota == a3, new_l2max, l3c)

            lax.fori_loop(0, _TOP_K, ext_body, l3)
            if prefetch:
                _dma(base + r + 1, 1, s0).start()

        _dma(base, 0, 0).start()
        _dma(base, 1, 1).start()
        _dma_l1(base, 0).start()

        def tri_body(t, c):
            r = t * 3
            do_row(r, 0, 1, 2, True)
            do_row(r + 1, 2, 0, 1, True)
            do_row(r + 2, 1, 2, 0, True)
            return c
        lax.fori_loop(0, (rpw - 1) // 3, tri_body, 0)
        do_row(rpw - 1, 0, 1, 2, False)
        pltpu.sync_copy(out_v, out_hbm.at[pl.ds(base, rpw)])

    return topk_kernel(sim, l1)


def _pool_kernel(p_ref, w1_ref, b1_ref, w2t_ref, b2_ref, o_ref):
    patches = p_ref[...]
    h = lax.dot_general(
        patches, w1_ref[...],
        dimension_numbers=(((1,), (0,)), ((), ())),
        preferred_element_type=jnp.float32,
    )
    h = jnp.maximum(h + b1_ref[...], 0.0)
    z = jnp.sum(h * w2t_ref[...], axis=1, keepdims=True) + b2_ref[...]
    z = z - jnp.max(z)
    w = jnp.exp(z)
    w = w / jnp.sum(w)
    gf = jnp.sum(patches * w, axis=0, keepdims=True)
    n = jnp.sqrt(jnp.sum(gf * gf))
    o_ref[...] = gf / jnp.maximum(n, 1e-12)


def _pool(patches, w1, b1, w2, b2):
    p, d = patches.shape
    dh = w1.shape[1]
    return pl.pallas_call(
        _pool_kernel,
        out_shape=jax.ShapeDtypeStruct((1, d), jnp.float32),
    )(patches, w1, b1.reshape(1, dh), w2.reshape(1, dh), b2.reshape(1, 1))


def kernel(test_patches, memory_nodes, W1, b1, W2, b2):
    p = test_patches.shape[0]
    sim, l1 = _sim_matmul(test_patches, memory_nodes)
    idx_pad = _topk_sc(sim, l1, 64)
    topk = idx_pad[:, :_TOP_K]

    memory_node_idx = topk.reshape(-1)
    test_node_idx = jnp.broadcast_to(
        jnp.arange(p, dtype=jnp.int32)[:, None], (p, _TOP_K)).reshape(-1)
    edge_index = jnp.stack([memory_node_idx, test_node_idx], axis=0)

    global_feature = _pool(test_patches, W1, b1, W2, b2)
    return edge_index, global_feature


# speedup vs baseline: 26.0214x; 1.0506x over previous
import functools

import jax
import jax.numpy as jnp
from jax import lax
from jax.experimental import pallas as pl
from jax.experimental.pallas import tpu as pltpu
from jax.experimental.pallas import tpu_sc as plsc

_TOP_K = 50
_LANES = 16


def _sim_matmul_kernel(a_ref, b_ref, o_ref, l1_ref):
    s = lax.dot_general(
        a_ref[...], b_ref[...],
        dimension_numbers=(((1,), (1,)), ((), ())),
        preferred_element_type=jnp.float32,
    )
    o_ref[...] = s
    mx = s[:, 0:128]
    for q in range(1, _LANES):
        mx = jnp.maximum(mx, s[:, q * 128:(q + 1) * 128])
    l1_ref[...] = mx


def _sim_matmul(patches, nodes):
    p, d = patches.shape
    m = nodes.shape[0]
    bm = 2048
    return pl.pallas_call(
        _sim_matmul_kernel,
        grid=(m // bm,),
        in_specs=[
            pl.BlockSpec((p, d), lambda j: (0, 0)),
            pl.BlockSpec((bm, d), lambda j: (j, 0)),
        ],
        out_specs=[
            pl.BlockSpec((p, bm), lambda j: (0, j)),
            pl.BlockSpec((p, 128), lambda j: (0, j)),
        ],
        out_shape=[
            jax.ShapeDtypeStruct((p, m), jnp.float32),
            jax.ShapeDtypeStruct((p, m // _LANES), jnp.float32),
        ],
    )(patches, nodes)


def _topk_sc(sim, l1, k_pad):
    p, m = sim.shape
    nw = 32
    rpw = p // nw
    nv = m // _LANES
    nl1 = nv // _LANES
    nl2 = nl1 // _LANES
    assert nl2 == _LANES
    qsz = m // 4
    nslots = 7
    assert rpw % nslots == 1
    phases = [[(4 * p + i) % nslots for i in range(4)] for p in range(nslots)]

    mesh = plsc.VectorSubcoreMesh(core_axis_name="c", subcore_axis_name="s")

    @functools.partial(
        pl.kernel,
        mesh=mesh,
        compiler_params=pltpu.CompilerParams(needs_layout_passes=False),
        out_type=jax.ShapeDtypeStruct((p, k_pad), jnp.int32),
        scratch_types=[
            pltpu.VMEM((nslots * qsz,), jnp.float32),
            pltpu.VMEM((nv,), jnp.float32),
            pltpu.VMEM((nl1,), jnp.float32),
            pltpu.VMEM((rpw, k_pad), jnp.int32),
        ] + [pltpu.SemaphoreType.DMA] * nslots,
    )
    def topk_kernel(sim_hbm, l1_hbm, out_hbm, big_v, l1_v, l2_v, out_v, *sems):
        cid = lax.axis_index("c")
        sid = lax.axis_index("s")
        wid = sid * 2 + cid
        base = wid * rpw
        iota = lax.iota(jnp.int32, _LANES)
        lane0 = iota == 0
        neg_inf_v = jnp.full((_LANES,), -jnp.inf, jnp.float32)

        def _full_i(x):
            return jnp.full((_LANES,), x, jnp.int32)

        def _full_f(x):
            return jnp.full((_LANES,), x, jnp.float32)

        def _dma(row, q, slot):
            return pltpu.make_async_copy(
                sim_hbm.at[row, pl.ds(q * qsz, qsz)],
                big_v.at[pl.ds(slot * qsz, qsz)],
                sems[slot])

        def do_row(r, slots, nxt):
            pltpu.sync_copy(l1_hbm.at[base + r], l1_v)

            def l2_body(i, c):
                mx = l1_v[pl.ds(i * 256, _LANES)]
                for j in range(1, _LANES):
                    mx = jnp.maximum(
                        mx, l1_v[pl.ds(i * 256 + j * 16, _LANES)])
                l2_v[pl.ds(i * 16, _LANES)] = mx
                return c
            lax.fori_loop(0, nl2, l2_body, 0)

            l3 = l2_v[pl.ds(0, _LANES)]
            for j in range(1, _LANES):
                l3 = jnp.maximum(l3, l2_v[pl.ds(j * 16, _LANES)])

            for q in range(4):
                _dma(base + r, q, slots[q]).wait()
            if nxt is not None:
                for q in range(3):
                    _dma(base + r + 1, q, nxt[q]).start()

            offs = [slots[q] * qsz - q * 8 * 2048 for q in range(4)]

            def ext_body(e, l3c):
                ks3, is3 = plsc.sort_key_val(l3c, iota, descending=True)
                a3 = is3[0]
                v2 = plsc.load_gather(l2_v, [iota * 16 + a3])
                ks2, is2 = plsc.sort_key_val(v2, iota, descending=True)
                j2 = is2[0]
                v1 = plsc.load_gather(
                    l1_v, [_full_i(j2 * 256 + a3) + iota * 16])
                ks1, is1 = plsc.sort_key_val(v1, iota, descending=True)
                j1 = is1[0]
                q1 = j2 * 256 + j1 * 16 + a3
                b = q1 // 128
                ql = q1 - b * 128
                qi = b // 8
                off = jnp.where(
                    qi < 2,
                    jnp.where(qi == 0, offs[0], offs[1]),
                    jnp.where(qi == 2, offs[2], offs[3]))
                pbase = off + b * 2048 + ql
                g = plsc.load_gather(big_v, [_full_i(pbase) + iota * 128])
                ks0, is0 = plsc.sort_key_val(g, iota, descending=True)
                j0 = is0[0]
                pos = b * 2048 + j0 * 128 + ql

                plsc.store_scatter(
                    out_v, [_full_i(r), _full_i(e)], _full_i(pos), mask=lane0)
                plsc.store_scatter(
                    big_v, [_full_i(pbase + j0 * 128)], neg_inf_v, mask=lane0)
                new_leaf = ks0[1]
                plsc.store_scatter(
                    l1_v, [_full_i(q1)], _full_f(new_leaf), mask=lane0)
                new_l1max = jnp.maximum(ks1[1], new_leaf)
                plsc.store_scatter(
                    l2_v, [_full_i(j2 * 16 + a3)], _full_f(new_l1max), mask=lane0)
                new_l2max = jnp.maximum(ks2[1], new_l1max)
                return jnp.where(iota == a3, new_l2max, l3c)

            lax.fori_loop(0, _TOP_K, ext_body, l3)
            if nxt is not None:
                _dma(base + r + 1, 3, nxt[3]).start()

        for q in range(4):
            _dma(base, q, q).start()

        def sept_body(t, c):
            r = t * nslots
            for pph in range(nslots):
                do_row(r + pph, phases[pph], phases[(pph + 1) % nslots])
            return c
        lax.fori_loop(0, (rpw - 1) // nslots, sept_body, 0)
        do_row(rpw - 1, phases[0], None)
        pltpu.sync_copy(out_v, out_hbm.at[pl.ds(base, rpw)])

    return topk_kernel(sim, l1)


def _pool_kernel(p_ref, w1_ref, b1_ref, w2t_ref, b2_ref, o_ref):
    patches = p_ref[...]
    h = lax.dot_general(
        patches, w1_ref[...],
        dimension_numbers=(((1,), (0,)), ((), ())),
        preferred_element_type=jnp.float32,
    )
    h = jnp.maximum(h + b1_ref[...], 0.0)
    z = jnp.sum(h * w2t_ref[...], axis=1, keepdims=True) + b2_ref[...]
    z = z - jnp.max(z)
    w = jnp.exp(z)
    w = w / jnp.sum(w)
    gf = jnp.sum(patches * w, axis=0, keepdims=True)
    n = jnp.sqrt(jnp.sum(gf * gf))
    o_ref[...] = gf / jnp.maximum(n, 1e-12)


def _pool(patches, w1, b1, w2, b2):
    p, d = patches.shape
    dh = w1.shape[1]
    return pl.pallas_call(
        _pool_kernel,
        out_shape=jax.ShapeDtypeStruct((1, d), jnp.float32),
    )(patches, w1, b1.reshape(1, dh), w2.reshape(1, dh), b2.reshape(1, 1))


def kernel(test_patches, memory_nodes, W1, b1, W2, b2):
    p = test_patches.shape[0]
    sim, l1 = _sim_matmul(test_patches, memory_nodes)
    idx_pad = _topk_sc(sim, l1, 64)
    topk = idx_pad[:, :_TOP_K]

    memory_node_idx = topk.reshape(-1)
    test_node_idx = jnp.broadcast_to(
        jnp.arange(p, dtype=jnp.int32)[:, None], (p, _TOP_K)).reshape(-1)
    edge_index = jnp.stack([memory_node_idx, test_node_idx], axis=0)

    global_feature = _pool(test_patches, W1, b1, W2, b2)
    return edge_index, global_feature
